# baseline (speedup 1.0000x reference)
import dataclasses
import functools

import jax
import jax.numpy as jnp
from jax import lax
from jax.experimental import pallas as pl
from jax.experimental.pallas import tpu as pltpu
from jax.experimental.pallas import tpu_sc as plsc

NC = 2
NS = 16
LN = 16
CH = 128


def _lane_splat(vec, i):
    idx = jnp.full((LN, 1), i, jnp.int32)
    dn = lax.GatherDimensionNumbers(
        offset_dims=(), collapsed_slice_dims=(0,), start_index_map=(0,))
    return lax.gather(vec, idx, dn, slice_sizes=(1,),
                      mode=lax.GatherScatterMode.PROMISE_IN_BOUNDS)


def _norm_body(deg_ref, lam_ref, nh_ref):
    lam = lam_ref[0, 0]
    dl = lam * deg_ref[...] + (1.0 - lam)
    nh_ref[...] = lax.rsqrt(dl)


def _combine_body(y_ref, x_ref, deg_ref, h_ref, alp_ref, lam_ref, o_ref):
    alp = alp_ref[0, 0]
    lam = lam_ref[0, 0]
    dl = lam * deg_ref[...] + (1.0 - lam)
    nh = lax.rsqrt(dl)
    agg = jnp.concatenate([h_ref[0], h_ref[1]], axis=1)
    o_ref[...] = ((1.0 - alp) * y_ref[...]
                  + (alp * lam) * (nh * agg)
                  + alp * (x_ref[...] / dl))


def _make_sc_kernel(n2, dh, chunks, npad):
    rows_per_tile = n2 // NS
    mesh = plsc.VectorSubcoreMesh(core_axis_name="c", subcore_axis_name="s")
    cp = pltpu.CompilerParams()
    for field, val in (("needs_layout_passes", False),
                       ("use_tc_tiling_on_sc", False)):
        if field in pltpu.CompilerParams.__dataclass_fields__:
            cp = dataclasses.replace(cp, **{field: val})

    @functools.partial(
        pl.kernel,
        mesh=mesh,
        compiler_params=cp,
        out_type=jax.ShapeDtypeStruct((NC, n2, dh), jnp.float32),
        scratch_types=[
            pltpu.VMEM((chunks, CH), jnp.int32),
            pltpu.VMEM((chunks, CH), jnp.int32),
            pltpu.VMEM((chunks, CH), jnp.float32),
            pltpu.VMEM((npad,), jnp.float32),
            pltpu.VMEM((2 * CH, dh), jnp.float32),
            pltpu.VMEM_SHARED((n2, dh), jnp.float32),
            pltpu.SemaphoreType.DMA,
            pltpu.SemaphoreType.DMA,
            pltpu.SemaphoreType.DMA,
            pltpu.SemaphoreType.DMA,
        ],
    )
    def sc_fn(yh, srcs, dsts, ws, nh, out,
              src_v, dst_v, w_v, nh_v, rows_v, acc,
              g0, g1, s0, s1):
        gsem = (g0, g1)
        ssem = (s0, s1)
        c = lax.axis_index("c")
        s = lax.axis_index("s")
        base = s * rows_per_tile

        pltpu.sync_copy(srcs.at[s], src_v)
        pltpu.sync_copy(dsts.at[s], dst_v)
        pltpu.sync_copy(ws.at[s], w_v)
        pltpu.sync_copy(nh, nh_v)

        @pl.loop(0, CH)
        def _zero_row(r):
            for j in range(dh // LN):
                rows_v[r, pl.ds(j * LN, LN)] = jnp.zeros((LN,), jnp.float32)

        n_full, rem = divmod(rows_per_tile, CH)
        for k in range(n_full):
            pltpu.sync_copy(rows_v.at[pl.ds(0, CH)], acc.at[pl.ds(base + k * CH, CH)])
        if rem:
            pltpu.sync_copy(rows_v.at[pl.ds(0, rem)],
                            acc.at[pl.ds(base + n_full * CH, rem)])

        plsc.subcore_barrier()

        def scale(b, ci):
            for g in range(CH // LN):
                sidx = src_v[ci, pl.ds(g * LN, LN)]
                wv = w_v[ci, pl.ds(g * LN, LN)]
                nh16 = plsc.load_gather(nh_v, [sidx])
                sv = wv * nh16
                for i in range(LN):
                    sp = _lane_splat(sv, i)
                    e = g * LN + i
                    for j in range(dh // LN):
                        slc = pl.ds(j * LN, LN)
                        rows_v[b * CH + e, slc] = rows_v[b * CH + e, slc] * sp

        def buf(b):
            return rows_v.at[pl.ds(b * CH, CH)]

        n_iter = chunks // 2
        for b in range(2):
            pltpu.async_copy(yh.at[c].at[src_v.at[b]], buf(b), gsem[b])

        @pl.loop(0, n_iter)
        def _ring(h):
            c0 = 2 * h
            for b in range(2):
                pltpu.make_async_copy(yh.at[c].at[src_v.at[c0 + b]],
                                      buf(b), gsem[b]).wait()
                scale(b, c0 + b)
                pltpu.async_copy(buf(b), acc.at[dst_v.at[c0 + b]],
                                 ssem[b], add=True)
            for b in range(2):
                pltpu.make_async_copy(buf(b), acc.at[dst_v.at[c0 + b]],
                                      ssem[b]).wait()

                @pl.when(h < n_iter - 1)
                def _prefetch():
                    pltpu.async_copy(yh.at[c].at[src_v.at[c0 + 2 + b]],
                                     buf(b), gsem[b])

        plsc.subcore_barrier()
        pltpu.sync_copy(acc.at[pl.ds(base, rows_per_tile)],
                        out.at[c, pl.ds(base, rows_per_tile)])

    return sc_fn


def kernel(Y, X, edge_weight, deg, alp, lam, edge_index):
    n, d = Y.shape
    e = edge_weight.shape[0]
    dh = d // 2
    chunks = 4 * (-(-e // (NS * CH * 4)))
    epad = NS * chunks * CH
    npad = -(-n // 128) * 128
    n2 = NS * 8 * (-(-n // (NS * 8)))

    src = edge_index[0].astype(jnp.int32)
    dst = edge_index[1].astype(jnp.int32)
    w = edge_weight.astype(jnp.float32)
    pad = epad - e
    if pad:
        src = jnp.concatenate([src, jnp.zeros((pad,), jnp.int32)])
        dst = jnp.concatenate([dst, jnp.zeros((pad,), jnp.int32)])
        w = jnp.concatenate([w, jnp.zeros((pad,), jnp.float32)])
    src3 = src.reshape(NS, chunks, CH)
    dst3 = dst.reshape(NS, chunks, CH)
    w3 = w.reshape(NS, chunks, CH)
    ypad = Y
    if n2 > n:
        ypad = jnp.concatenate([Y, jnp.zeros((n2 - n, d), jnp.float32)])
    yh = jnp.stack([ypad[:, :dh], ypad[:, dh:]])

    deg_pad = deg
    if npad > n:
        deg_pad = jnp.concatenate([deg, jnp.ones((npad - n,), jnp.float32)])
    lam11 = lam.reshape(1, 1)
    alp11 = alp.reshape(1, 1)

    nh_pad = pl.pallas_call(
        _norm_body,
        out_shape=jax.ShapeDtypeStruct((npad // 128, 128), jnp.float32),
    )(deg_pad.reshape(npad // 128, 128), lam11)
    nh_flat = nh_pad.reshape(npad)

    halves = _make_sc_kernel(n2, dh, chunks, npad)(yh, src3, dst3, w3,
                                                   nh_flat)[:, :n, :]

    blk = 2000
    out = pl.pallas_call(
        _combine_body,
        grid=(n // blk,),
        in_specs=[
            pl.BlockSpec((blk, d), lambda i: (i, 0)),
            pl.BlockSpec((blk, d), lambda i: (i, 0)),
            pl.BlockSpec((blk, 1), lambda i: (i, 0)),
            pl.BlockSpec((NC, blk, dh), lambda i: (0, i, 0)),
            pl.BlockSpec((1, 1), lambda i: (0, 0)),
            pl.BlockSpec((1, 1), lambda i: (0, 0)),
        ],
        out_specs=pl.BlockSpec((blk, d), lambda i: (i, 0)),
        out_shape=jax.ShapeDtypeStruct((n, d), jnp.float32),
    )(Y, X, deg[:, None], halves, alp11, lam11)
    return out


# speedup vs baseline: 1.3467x; 1.3467x over previous
import dataclasses
import functools

import jax
import jax.numpy as jnp
from jax import lax
from jax.experimental import pallas as pl
from jax.experimental.pallas import tpu as pltpu
from jax.experimental.pallas import tpu_sc as plsc

NC = 2
NS = 16
LN = 16
CH = 128


def _lane_splat(vec, i):
    idx = jnp.full((LN, 1), i, jnp.int32)
    dn = lax.GatherDimensionNumbers(
        offset_dims=(), collapsed_slice_dims=(0,), start_index_map=(0,))
    return lax.gather(vec, idx, dn, slice_sizes=(1,),
                      mode=lax.GatherScatterMode.PROMISE_IN_BOUNDS)


def _scale_y_body(y_ref, deg_ref, lam_ref, h_ref):
    lam = lam_ref[0, 0]
    dl = lam * deg_ref[...] + (1.0 - lam)
    yp = y_ref[...] * lax.rsqrt(dl)
    dh = y_ref.shape[1] // 2
    h_ref[0] = yp[:, :dh]
    h_ref[1] = yp[:, dh:]


def _combine_body(y_ref, x_ref, deg_ref, h_ref, alp_ref, lam_ref, o_ref):
    alp = alp_ref[0, 0]
    lam = lam_ref[0, 0]
    dl = lam * deg_ref[...] + (1.0 - lam)
    nh = lax.rsqrt(dl)
    agg = jnp.concatenate([h_ref[0], h_ref[1]], axis=1)
    o_ref[...] = ((1.0 - alp) * y_ref[...]
                  + (alp * lam) * (nh * agg)
                  + alp * (x_ref[...] / dl))


def _make_sc_kernel(n2, dh, chunks):
    rows_per_tile = n2 // NS
    mesh = plsc.VectorSubcoreMesh(core_axis_name="c", subcore_axis_name="s")
    cp = pltpu.CompilerParams()
    for field, val in (("needs_layout_passes", False),
                       ("use_tc_tiling_on_sc", False)):
        if field in pltpu.CompilerParams.__dataclass_fields__:
            cp = dataclasses.replace(cp, **{field: val})

    @functools.partial(
        pl.kernel,
        mesh=mesh,
        compiler_params=cp,
        out_type=jax.ShapeDtypeStruct((NC, n2, dh), jnp.float32),
        scratch_types=[
            pltpu.VMEM((chunks, CH), jnp.int32),
            pltpu.VMEM((chunks, CH), jnp.int32),
            pltpu.VMEM((chunks, CH), jnp.float32),
            pltpu.VMEM((3 * CH, dh), jnp.float32),
            pltpu.VMEM_SHARED((n2, dh), jnp.float32),
            pltpu.SemaphoreType.DMA,
            pltpu.SemaphoreType.DMA,
            pltpu.SemaphoreType.DMA,
            pltpu.SemaphoreType.DMA,
            pltpu.SemaphoreType.DMA,
            pltpu.SemaphoreType.DMA,
        ],
    )
    def sc_fn(yh, srcs, dsts, ws, out,
              src_v, dst_v, w_v, rows_v, acc,
              g0, g1, g2, s0, s1, s2):
        gsem = (g0, g1, g2)
        ssem = (s0, s1, s2)
        c = lax.axis_index("c")
        s = lax.axis_index("s")
        base = s * rows_per_tile

        pltpu.sync_copy(srcs.at[s], src_v)
        pltpu.sync_copy(dsts.at[s], dst_v)
        pltpu.sync_copy(ws.at[s], w_v)

        @pl.loop(0, 3 * CH)
        def _zero_row(r):
            for j in range(dh // LN):
                rows_v[r, pl.ds(j * LN, LN)] = jnp.zeros((LN,), jnp.float32)

        n_full, rem = divmod(rows_per_tile, CH)
        for k in range(n_full):
            pltpu.sync_copy(rows_v.at[pl.ds(0, CH)], acc.at[pl.ds(base + k * CH, CH)])
        if rem:
            pltpu.sync_copy(rows_v.at[pl.ds(0, rem)],
                            acc.at[pl.ds(base + n_full * CH, rem)])

        plsc.subcore_barrier()

        def scale(b, ci):
            for g in range(CH // LN):
                wv = w_v[ci, pl.ds(g * LN, LN)]
                for i in range(LN):
                    sp = _lane_splat(wv, i)
                    e = g * LN + i
                    for j in range(dh // LN):
                        slc = pl.ds(j * LN, LN)
                        rows_v[b * CH + e, slc] = rows_v[b * CH + e, slc] * sp

        def buf(b):
            return rows_v.at[pl.ds(b * CH, CH)]

        def g_start(b, ci):
            pltpu.async_copy(yh.at[c].at[src_v.at[ci]], buf(b), gsem[b])

        def g_wait(b, ci):
            pltpu.make_async_copy(yh.at[c].at[src_v.at[ci]], buf(b),
                                  gsem[b]).wait()

        def s_start(b, ci):
            pltpu.async_copy(buf(b), acc.at[dst_v.at[ci]], ssem[b], add=True)

        def s_wait(b, ci):
            pltpu.make_async_copy(buf(b), acc.at[dst_v.at[ci]],
                                  ssem[b]).wait()

        g_start(0, 0)
        s_start(1, 0)
        s_start(2, 0)

        n_iter = chunks // 3
        @pl.loop(0, n_iter)
        def _ring(h):
            c0 = 3 * h
            for b in range(3):
                ci = c0 + b
                bn = (b + 1) % 3
                s_wait(bn, ci)
                if b < 2:
                    g_start(bn, ci + 1)
                else:
                    @pl.when(h < n_iter - 1)
                    def _prefetch():
                        g_start(bn, ci + 1)
                g_wait(b, ci)
                scale(b, ci)
                s_start(b, ci)

        s_wait((chunks - 2) % 3, chunks - 2)
        s_wait((chunks - 1) % 3, chunks - 1)

        plsc.subcore_barrier()
        pltpu.sync_copy(acc.at[pl.ds(base, rows_per_tile)],
                        out.at[c, pl.ds(base, rows_per_tile)])

    return sc_fn


def kernel(Y, X, edge_weight, deg, alp, lam, edge_index):
    n, d = Y.shape
    e = edge_weight.shape[0]
    dh = d // 2
    chunks = 3 * (-(-e // (NS * CH * 3)))
    epad = NS * chunks * CH
    n2 = NS * 8 * (-(-n // (NS * 8)))

    src = edge_index[0].astype(jnp.int32)
    dst = edge_index[1].astype(jnp.int32)
    w = edge_weight.astype(jnp.float32)
    pad = epad - e
    if pad:
        src = jnp.concatenate([src, jnp.zeros((pad,), jnp.int32)])
        dst = jnp.concatenate([dst, jnp.zeros((pad,), jnp.int32)])
        w = jnp.concatenate([w, jnp.zeros((pad,), jnp.float32)])
    src3 = src.reshape(NS, chunks, CH)
    dst3 = dst.reshape(NS, chunks, CH)
    w3 = w.reshape(NS, chunks, CH)
    ypad = Y
    deg_pad = deg
    if n2 > n:
        ypad = jnp.concatenate([Y, jnp.zeros((n2 - n, d), jnp.float32)])
        deg_pad = jnp.concatenate([deg, jnp.ones((n2 - n,), jnp.float32)])
    lam11 = lam.reshape(1, 1)
    alp11 = alp.reshape(1, 1)

    yh = pl.pallas_call(
        _scale_y_body,
        out_shape=jax.ShapeDtypeStruct((NC, n2, dh), jnp.float32),
    )(ypad, deg_pad[:, None], lam11)

    halves = _make_sc_kernel(n2, dh, chunks)(yh, src3, dst3, w3)[:, :n, :]

    blk = 2000
    out = pl.pallas_call(
        _combine_body,
        grid=(n // blk,),
        in_specs=[
            pl.BlockSpec((blk, d), lambda i: (i, 0)),
            pl.BlockSpec((blk, d), lambda i: (i, 0)),
            pl.BlockSpec((blk, 1), lambda i: (i, 0)),
            pl.BlockSpec((NC, blk, dh), lambda i: (0, i, 0)),
            pl.BlockSpec((1, 1), lambda i: (0, 0)),
            pl.BlockSpec((1, 1), lambda i: (0, 0)),
        ],
        out_specs=pl.BlockSpec((blk, d), lambda i: (i, 0)),
        out_shape=jax.ShapeDtypeStruct((n, d), jnp.float32),
    )(Y, X, deg[:, None], halves, alp11, lam11)
    return out


# speedup vs baseline: 1.5457x; 1.1478x over previous
import dataclasses
import functools

import jax
import jax.numpy as jnp
from jax import lax
from jax.experimental import pallas as pl
from jax.experimental.pallas import tpu as pltpu
from jax.experimental.pallas import tpu_sc as plsc

NC = 2
NS = 16
LN = 16
CH = 128

_PERM32 = [0, 16, 1, 17, 2, 18, 3, 19, 4, 20, 5, 21, 6, 22, 7, 23,
           8, 24, 9, 25, 10, 26, 11, 27, 12, 28, 13, 29, 14, 30, 15, 31]


def _lane_splat(vec, i):
    idx = jnp.full((LN, 1), i, jnp.int32)
    dn = lax.GatherDimensionNumbers(
        offset_dims=(), collapsed_slice_dims=(0,), start_index_map=(0,))
    return lax.gather(vec, idx, dn, slice_sizes=(1,),
                      mode=lax.GatherScatterMode.PROMISE_IN_BOUNDS)


def _scale_y_body(y_ref, deg_ref, lam_ref, h_ref):
    lam = lam_ref[0, 0]
    dl = lam * deg_ref[...] + (1.0 - lam)
    yp = (y_ref[...] * lax.rsqrt(dl)).astype(jnp.bfloat16)
    dh = y_ref.shape[1] // 2
    h_ref[0] = yp[:, :dh]
    h_ref[1] = yp[:, dh:]


def _combine_body(y_ref, x_ref, deg_ref, h_ref, alp_ref, lam_ref, o_ref):
    alp = alp_ref[0, 0]
    lam = lam_ref[0, 0]
    dl = lam * deg_ref[...] + (1.0 - lam)
    nh = lax.rsqrt(dl)
    agg = jnp.concatenate([h_ref[0], h_ref[1]], axis=1)
    o_ref[...] = ((1.0 - alp) * y_ref[...]
                  + (alp * lam) * (nh * agg)
                  + alp * (x_ref[...] / dl))


def _make_sc_kernel(n2, dh, chunks):
    rows_per_tile = n2 // NS
    mesh = plsc.VectorSubcoreMesh(core_axis_name="c", subcore_axis_name="s")
    cp = pltpu.CompilerParams()
    for field, val in (("needs_layout_passes", False),
                       ("use_tc_tiling_on_sc", False)):
        if field in pltpu.CompilerParams.__dataclass_fields__:
            cp = dataclasses.replace(cp, **{field: val})

    @functools.partial(
        pl.kernel,
        mesh=mesh,
        compiler_params=cp,
        out_type=jax.ShapeDtypeStruct((NC, n2, dh), jnp.float32),
        scratch_types=[
            pltpu.VMEM((chunks, CH), jnp.int32),
            pltpu.VMEM((chunks, CH), jnp.int32),
            pltpu.VMEM((chunks, CH), jnp.float32),
            pltpu.VMEM((2 * CH, dh), jnp.bfloat16),
            pltpu.VMEM((2 * CH, dh), jnp.float32),
            pltpu.SemaphoreType.DMA,
            pltpu.SemaphoreType.DMA,
            pltpu.SemaphoreType.DMA,
            pltpu.SemaphoreType.DMA,
            pltpu.VMEM_SHARED((n2, dh), jnp.float32),
        ],
    )
    def sc_fn(yh, srcs, dsts, ws, out,
              src_v, dst_v, w_v, grows_v, frows_v,
              g0, g1, s0, s1, acc):
        gsem = (g0, g1)
        ssem = (s0, s1)
        c = lax.axis_index("c")
        s = lax.axis_index("s")
        base = s * rows_per_tile

        pltpu.sync_copy(srcs.at[s], src_v)
        pltpu.sync_copy(dsts.at[s], dst_v)
        pltpu.sync_copy(ws.at[s], w_v)

        @pl.loop(0, 2 * CH)
        def _zero_row(r):
            for j in range(dh // LN):
                frows_v[r, pl.ds(j * LN, LN)] = jnp.zeros((LN,), jnp.float32)

        n_full, rem = divmod(rows_per_tile, CH)
        for k in range(n_full):
            pltpu.sync_copy(frows_v.at[pl.ds(0, CH)],
                            acc.at[pl.ds(base + k * CH, CH)])
        if rem:
            pltpu.sync_copy(frows_v.at[pl.ds(0, rem)],
                            acc.at[pl.ds(base + n_full * CH, rem)])

        plsc.subcore_barrier()

        def gbuf(b):
            return grows_v.at[pl.ds(b * CH, CH)]

        def fbuf(b):
            return frows_v.at[pl.ds(b * CH, CH)]

        def g_start(b, ci):
            pltpu.async_copy(yh.at[c].at[src_v.at[ci]], gbuf(b), gsem[b])

        def g_wait(b, ci):
            pltpu.make_async_copy(yh.at[c].at[src_v.at[ci]], gbuf(b),
                                  gsem[b]).wait()

        def s_start(b, ci):
            pltpu.async_copy(fbuf(b), acc.at[dst_v.at[ci]], ssem[b], add=True)

        def s_wait(b, ci):
            pltpu.make_async_copy(fbuf(b), acc.at[dst_v.at[ci]],
                                  ssem[b]).wait()

        def scale(b, ci):
            for g in range(CH // LN):
                wv = w_v[ci, pl.ds(g * LN, LN)]
                for i in range(LN):
                    sp = _lane_splat(wv, i)
                    e = g * LN + i
                    for g2 in range(dh // 32):
                        packed = grows_v[b * CH + e, pl.ds(g2 * 32, 32)]
                        lo, hi = plsc.unpack(
                            packed, format=plsc.PackFormat.INTERLEAVED)
                        frows_v[b * CH + e, pl.ds(g2 * 32, LN)] = lo * sp
                        frows_v[b * CH + e, pl.ds(g2 * 32 + LN, LN)] = hi * sp

        g_start(0, 0)
        s_start(0, 0)
        s_start(1, 0)

        n_iter = chunks // 2

        @pl.loop(0, n_iter)
        def _ring(h):
            c0 = 2 * h
            for b in range(2):
                ci = c0 + b
                bn = 1 - b
                g_wait(b, ci)
                if b == 0:
                    g_start(bn, ci + 1)
                else:
                    @pl.when(h < n_iter - 1)
                    def _prefetch():
                        g_start(bn, ci + 1)
                s_wait(b, ci)
                scale(b, ci)
                s_start(b, ci)

        s_wait(0, chunks - 2)
        s_wait(1, chunks - 1)

        plsc.subcore_barrier()
        pltpu.sync_copy(acc.at[pl.ds(base, rows_per_tile)],
                        out.at[c, pl.ds(base, rows_per_tile)])

    return sc_fn


def kernel(Y, X, edge_weight, deg, alp, lam, edge_index):
    n, d = Y.shape
    e = edge_weight.shape[0]
    dh = d // 2
    chunks = 2 * (-(-e // (NS * CH * 2)))
    epad = NS * chunks * CH
    n2 = NS * 8 * (-(-n // (NS * 8)))

    src = edge_index[0].astype(jnp.int32)
    dst = edge_index[1].astype(jnp.int32)
    w = edge_weight.astype(jnp.float32)
    pad = epad - e
    if pad:
        src = jnp.concatenate([src, jnp.zeros((pad,), jnp.int32)])
        dst = jnp.concatenate([dst, jnp.zeros((pad,), jnp.int32)])
        w = jnp.concatenate([w, jnp.zeros((pad,), jnp.float32)])
    src3 = src.reshape(NS, chunks, CH)
    dst3 = dst.reshape(NS, chunks, CH)
    w3 = w.reshape(NS, chunks, CH)
    ypad = Y
    deg_pad = deg
    if n2 > n:
        ypad = jnp.concatenate([Y, jnp.zeros((n2 - n, d), jnp.float32)])
        deg_pad = jnp.concatenate([deg, jnp.ones((n2 - n,), jnp.float32)])
    lam11 = lam.reshape(1, 1)
    alp11 = alp.reshape(1, 1)

    yh = pl.pallas_call(
        _scale_y_body,
        out_shape=jax.ShapeDtypeStruct((NC, n2, dh), jnp.bfloat16),
    )(ypad, deg_pad[:, None], lam11)
    if _PERM32 != list(range(32)):
        perm = jnp.asarray([g * 32 + p for g in range(dh // 32)
                            for p in _PERM32], dtype=jnp.int32)
        yh = yh[:, :, perm]

    halves = _make_sc_kernel(n2, dh, chunks)(yh, src3, dst3, w3)[:, :n, :]

    blk = 2000
    out = pl.pallas_call(
        _combine_body,
        grid=(n // blk,),
        in_specs=[
            pl.BlockSpec((blk, d), lambda i: (i, 0)),
            pl.BlockSpec((blk, d), lambda i: (i, 0)),
            pl.BlockSpec((blk, 1), lambda i: (i, 0)),
            pl.BlockSpec((NC, blk, dh), lambda i: (0, i, 0)),
            pl.BlockSpec((1, 1), lambda i: (0, 0)),
            pl.BlockSpec((1, 1), lambda i: (0, 0)),
        ],
        out_specs=pl.BlockSpec((blk, d), lambda i: (i, 0)),
        out_shape=jax.ShapeDtypeStruct((n, d), jnp.float32),
    )(Y, X, deg[:, None], halves, alp11, lam11)
    return out
